# Initial kernel scaffold; baseline (speedup 1.0000x reference)
import jax
import jax.numpy as jnp
from jax.experimental import pallas as pl


def kernel(fire_signal, acoustic_src, conv_w, ln_gamma, ln_beta, dense_w, dense_b, proj_w, proj_b, comb_w, comb_b, input_lengths, target_lengths):
    raise NotImplementedError("write your pallas kernel here")


# speedup vs baseline: 67.1467x; 67.1467x over previous
import functools

import jax
import jax.numpy as jnp
from jax.experimental import pallas as pl
from jax.experimental.pallas import tpu as pltpu

F32 = jnp.float32
_B = 8
_T = 4096
_D = 128
_TSW = 512
_ACO = 192
_NF = 128
_DM = 512
_FOLD = _T // _D


def _softplus(x):
    return jnp.maximum(x, 0.0) + jnp.log(1.0 + jnp.exp(-jnp.abs(x)))


def _sigmoid(x):
    t = jnp.exp(-jnp.abs(x))
    return jnp.where(x >= 0, 1.0 / (1.0 + t), t / (1.0 + t))


def _cif_body(wk_ref, gamma_ref, beta_ref, dwT_ref, db_ref, pwT_ref, pb_ref,
              cw1T_ref, cw2T_ref, cb_ref, il_ref, tl_ref,
              x_ref, src_ref,
              emb_ref, alpha_ref, q_ref, interpret=False):
    b = pl.program_id(0)
    x = x_ref[0]
    z1 = jnp.zeros((1, _D), F32)
    xm1 = jnp.concatenate([z1, x[:-1]], axis=0)
    xp1 = jnp.concatenate([x[1:], z1], axis=0)
    h0 = xm1 * wk_ref[0:1] + x * wk_ref[1:2] + xp1 * wk_ref[2:3] + x
    mu = jnp.mean(h0, axis=1, keepdims=True)
    var = jnp.mean((h0 - mu) ** 2, axis=1, keepdims=True)
    xn = (h0 - mu) / jnp.sqrt(var + 1e-5) * gamma_ref[0:1] + beta_ref[0:1]
    h = jnp.maximum(
        jax.lax.dot_general(xn, dwT_ref[...], (((1,), (0,)), ((), ())),
                            precision=jax.lax.Precision.HIGHEST) + db_ref[0:1],
        0.0)
    z = jax.lax.dot_general(h, pwT_ref[...], (((1,), (0,)), ((), ())),
                            precision=jax.lax.Precision.HIGHEST) + pb_ref[0, 0]
    zf = jnp.reshape(z, (_FOLD, _D))
    alpha_raw = _softplus(zf)
    onset = _sigmoid((alpha_raw - 1.0) * 10.0)
    ilen = il_ref[b]
    tf = (jax.lax.broadcasted_iota(jnp.int32, (_FOLD, _D), 0) * _D
          + jax.lax.broadcasted_iota(jnp.int32, (_FOLD, _D), 1))
    pad = tf >= ilen
    alpha = jnp.where(pad, 0.0, alpha_raw)
    onset = jnp.where(pad, 0.0, onset)
    alpha_ref[0] = alpha
    contrib = jnp.full((1, 1), jnp.abs(jnp.sum(onset) - tl_ref[b].astype(F32))
                       * (1.0 / _B), F32)

    @pl.when(b == 0)
    def _():
        q_ref[...] = contrib

    @pl.when(b > 0)
    def _():
        q_ref[...] += contrib

    m = (alpha > 1.0).astype(F32)
    r_iota = jax.lax.broadcasted_iota(jnp.int32, (_D, _D), 0)
    c_iota = jax.lax.broadcasted_iota(jnp.int32, (_D, _D), 1)
    U = (r_iota <= c_iota).astype(F32)
    row_cum = jax.lax.dot_general(m, U, (((1,), (0,)), ((), ())),
                                  precision=jax.lax.Precision.HIGHEST)
    r_tot = row_cum[:, _D - 1:_D]
    ls_r = jax.lax.broadcasted_iota(jnp.int32, (_FOLD, _FOLD), 0)
    ls_c = jax.lax.broadcasted_iota(jnp.int32, (_FOLD, _FOLD), 1)
    Ls = (ls_c < ls_r).astype(F32)
    off = jax.lax.dot_general(Ls, r_tot, (((1,), (0,)), ((), ())),
                              precision=jax.lax.Precision.HIGHEST)
    cums = row_cum + off
    n_fires = jnp.sum(m)
    slots = (jax.lax.broadcasted_iota(jnp.int32, (1, 1, _NF), 2) + 1).astype(F32)
    cmp = (cums[:, :, None] < slots).astype(F32)
    ff_f = jnp.sum(cmp, axis=(0, 1))
    ff = jnp.clip(ff_f.astype(jnp.int32), 0, _T - 1).reshape(1, _NF)
    t_col = jax.lax.broadcasted_iota(jnp.int32, (_T, 1), 0)
    oh_bi = (t_col == ff).astype(F32)
    temporal = jax.lax.dot_general(oh_bi, x, (((0,), (0,)), ((), ())),
                                   precision=jax.lax.Precision.HIGHEST)
    valid = (jax.lax.broadcasted_iota(jnp.int32, (_NF, 1), 0)
             < n_fires.astype(jnp.int32)).astype(F32)
    temporal = temporal * valid
    ffs0 = jnp.clip(ff // 8, 0, _TSW - 1)
    w_col = jax.lax.broadcasted_iota(jnp.int32, (_TSW, 1), 0)
    oh_sw = (w_col == ffs0).astype(F32)
    pitch = jax.lax.dot_general(oh_sw, src_ref[0], (((0,), (0,)), ((), ())),
                                precision=jax.lax.Precision.HIGHEST)
    emb = (jax.lax.dot_general(temporal, cw1T_ref[...], (((1,), (0,)), ((), ())),
                               precision=jax.lax.Precision.HIGHEST)
           + jax.lax.dot_general(pitch, cw2T_ref[...], (((1,), (0,)), ((), ())),
                                 precision=jax.lax.Precision.HIGHEST)
           + cb_ref[0:1])
    emb_ref[0] = emb


@functools.partial(jax.jit, static_argnames=("interpret",))
def _cif_run(fire_signal, acoustic_src, conv_w, ln_gamma, ln_beta, dense_w,
             dense_b, proj_w, proj_b, comb_w, comb_b, input_lengths,
             target_lengths, interpret=False):
    wk = jnp.zeros((8, _D), F32).at[0:3].set(jnp.transpose(conv_w[:, 0, :], (1, 0)))
    gamma = ln_gamma.reshape(1, _D)
    beta = ln_beta.reshape(1, _D)
    dwT = jnp.transpose(dense_w, (1, 0))
    db = dense_b.reshape(1, _D)
    pwT = jnp.transpose(proj_w, (1, 0))
    pb = proj_b.reshape(1, 1)
    cw1T = jnp.transpose(comb_w[:, :_D], (1, 0))
    cw2T = jnp.transpose(comb_w[:, _D:], (1, 0))
    cb = comb_b.reshape(1, _DM)

    full = lambda shape: pl.BlockSpec(shape, lambda b: (0,) * len(shape))
    grid_spec = pltpu.PrefetchScalarGridSpec(
        num_scalar_prefetch=0,
        grid=(_B,),
        in_specs=[
            full((8, _D)), full((1, _D)), full((1, _D)), full((_D, _D)),
            full((1, _D)), full((_D, 1)), full((1, 1)),
            full((_D, _DM)), full((_ACO, _DM)), full((1, _DM)),
            pl.BlockSpec(memory_space=pltpu.SMEM),
            pl.BlockSpec(memory_space=pltpu.SMEM),
            pl.BlockSpec((1, _T, _D), lambda b: (b, 0, 0)),
            pl.BlockSpec((1, _TSW, _ACO), lambda b: (b, 0, 0)),
        ],
        out_specs=[
            pl.BlockSpec((1, _NF, _DM), lambda b: (b, 0, 0)),
            pl.BlockSpec((1, _FOLD, _D), lambda b: (b, 0, 0)),
            pl.BlockSpec((1, 1), lambda b: (0, 0)),
        ],
    )
    embs, alpha_f, q = pl.pallas_call(
        functools.partial(_cif_body, interpret=interpret),
        grid_spec=grid_spec,
        out_shape=[
            jax.ShapeDtypeStruct((_B, _NF, _DM), F32),
            jax.ShapeDtypeStruct((_B, _FOLD, _D), F32),
            jax.ShapeDtypeStruct((1, 1), F32),
        ],
        interpret=interpret,
    )(wk, gamma, beta, dwT, db, pwT, pb, cw1T, cw2T, cb,
      input_lengths, target_lengths, fire_signal, acoustic_src)
    return embs, alpha_f.reshape(_B, _T), q[0, 0]


def kernel(fire_signal, acoustic_src, conv_w, ln_gamma, ln_beta, dense_w,
           dense_b, proj_w, proj_b, comb_w, comb_b, input_lengths,
           target_lengths):
    return _cif_run(fire_signal, acoustic_src, conv_w, ln_gamma, ln_beta,
                    dense_w, dense_b, proj_w, proj_b, comb_w, comb_b,
                    input_lengths, target_lengths)


# speedup vs baseline: 93.0002x; 1.3850x over previous
import functools

import jax
import jax.numpy as jnp
from jax import lax
from jax.experimental import pallas as pl
from jax.experimental.pallas import tpu as pltpu
from jax.experimental.pallas import tpu_sc as plsc

F32 = jnp.float32
I32 = jnp.int32
_B = 8
_T = 4096
_D = 128
_TSW = 512
_ACO = 192
_NF = 128
_DM = 512
_FOLD = _T // _D
_NC = 2
_NS = 16
_NW = _NC * _NS
_NROWS = _B * _NF
_RPW = _NROWS // _NW
_ACOP = 256


def _softplus(x):
    return jnp.maximum(x, 0.0) + jnp.log(1.0 + jnp.exp(-jnp.abs(x)))


def _sigmoid(x):
    t = jnp.exp(-jnp.abs(x))
    return jnp.where(x >= 0, 1.0 / (1.0 + t), t / (1.0 + t))


def _alpha_body(wk_ref, dwT_ref, db_ref, pw_ref, pb_ref,
                il_ref, tl_ref, x_ref,
                alpha_ref, q_ref, gff_ref, gffs_ref, valid_ref):
    b = pl.program_id(0)
    x = x_ref[0]
    z1 = jnp.zeros((1, _D), F32)
    xm1 = jnp.concatenate([z1, x[:-1]], axis=0)
    xp1 = jnp.concatenate([x[1:], z1], axis=0)
    h0 = xm1 * wk_ref[0:1] + x * wk_ref[1:2] + xp1 * wk_ref[2:3]
    mu = jnp.mean(h0, axis=1, keepdims=True)
    diff = h0 - mu
    var = jnp.mean(diff * diff, axis=1, keepdims=True)
    inv = jnp.reshape(1.0 / jnp.sqrt(jnp.reshape(var, (_FOLD, _D)) + 1e-5),
                      (_T, 1))
    xn = diff * inv
    h = jnp.maximum(
        lax.dot_general(xn, dwT_ref[...], (((1,), (0,)), ((), ())),
                        precision=lax.Precision.HIGHEST) + db_ref[0:1],
        0.0)
    z = jnp.sum(h * pw_ref[0:1], axis=1, keepdims=True) + pb_ref[0, 0]
    zf = jnp.reshape(z, (_FOLD, _D))
    alpha_raw = _softplus(zf)
    onset = _sigmoid((alpha_raw - 1.0) * 10.0)
    ilen = il_ref[b]
    tf = (lax.broadcasted_iota(I32, (_FOLD, _D), 0) * _D
          + lax.broadcasted_iota(I32, (_FOLD, _D), 1))
    pad = tf >= ilen
    alpha = jnp.where(pad, 0.0, alpha_raw)
    onset = jnp.where(pad, 0.0, onset)
    alpha_ref[0] = alpha
    contrib = jnp.full((1, 1), jnp.abs(jnp.sum(onset) - tl_ref[b].astype(F32))
                       * (1.0 / _B), F32)

    @pl.when(b == 0)
    def _():
        q_ref[...] = contrib

    @pl.when(b > 0)
    def _():
        q_ref[...] += contrib

    m = (alpha > 1.0).astype(F32)
    r_iota = lax.broadcasted_iota(I32, (_D, _D), 0)
    c_iota = lax.broadcasted_iota(I32, (_D, _D), 1)
    U = (r_iota <= c_iota).astype(F32)
    row_cum = lax.dot_general(m, U, (((1,), (0,)), ((), ())))
    r_tot = row_cum[:, _D - 1:_D]
    ls_r = lax.broadcasted_iota(I32, (_FOLD, _FOLD), 0)
    ls_c = lax.broadcasted_iota(I32, (_FOLD, _FOLD), 1)
    Ls = (ls_c < ls_r).astype(F32)
    off = lax.dot_general(Ls, r_tot, (((1,), (0,)), ((), ())))
    cums = row_cum + off
    n_fires = jnp.sum(m)
    slots = (lax.broadcasted_iota(I32, (1, 1, _NF), 2) + 1).astype(F32)
    cmp = (cums[:, :, None] < slots).astype(F32)
    ff_f = jnp.sum(cmp, axis=(0, 1))
    ff = jnp.clip(ff_f.astype(I32), 0, _T - 1).reshape(1, _NF)
    gff_ref[0] = ff + b * _T
    ffs0 = jnp.clip(ff // 8, 0, _TSW - 1)
    gffs_ref[0] = ffs0 + b * _TSW
    valid_ref[0] = (lax.broadcasted_iota(I32, (_NF, 1), 0)
                    < n_fires.astype(I32)).astype(F32)


def _sc_gather_body(gff_hbm, gffs_hbm, fire_hbm, src_hbm, out_t, out_p,
                    idx1_v, idx2_v, rows1_v, rows2_v, sem1, sem2):
    wid = lax.axis_index("s") * _NC + lax.axis_index("c")
    base = wid * _RPW
    pltpu.sync_copy(gff_hbm.at[pl.ds(base, _RPW)], idx1_v)
    pltpu.sync_copy(gffs_hbm.at[pl.ds(base, _RPW)], idx2_v)
    c1 = pltpu.async_copy(fire_hbm.at[idx1_v], rows1_v, sem1)
    c2 = pltpu.async_copy(src_hbm.at[idx2_v], rows2_v, sem2)
    c1.wait()
    c2.wait()
    pltpu.sync_copy(rows1_v, out_t.at[pl.ds(base, _RPW)])
    pltpu.sync_copy(rows2_v, out_p.at[pl.ds(base, _RPW)])


def _comb_body(cw1T_ref, cw2T_ref, cb_ref, t_ref, p_ref, v_ref, emb_ref):
    temporal = t_ref[0] * v_ref[0]
    emb_ref[0] = (lax.dot_general(temporal, cw1T_ref[...],
                                  (((1,), (0,)), ((), ())))
                  + lax.dot_general(p_ref[0], cw2T_ref[...],
                                    (((1,), (0,)), ((), ())))
                  + cb_ref[0:1])


@jax.jit
def _cif_run(fire_signal, acoustic_src, conv_w, ln_gamma, ln_beta, dense_w,
             dense_b, proj_w, proj_b, comb_w, comb_b, input_lengths,
             target_lengths):
    wk = jnp.zeros((8, _D), F32).at[0:3].set(
        jnp.transpose(conv_w[:, 0, :], (1, 0))).at[1].add(1.0)
    dwT = ln_gamma[:, None] * jnp.transpose(dense_w, (1, 0))
    db = (ln_beta @ jnp.transpose(dense_w, (1, 0)) + dense_b).reshape(1, _D)
    pw = proj_w.reshape(1, _D)
    pb = proj_b.reshape(1, 1)
    cw1T = jnp.transpose(comb_w[:, :_D], (1, 0))
    cw2T = jnp.zeros((_ACOP, _DM), F32).at[:_ACO].set(
        jnp.transpose(comb_w[:, _D:], (1, 0)))
    cb = comb_b.reshape(1, _DM)

    full = lambda shape: pl.BlockSpec(shape, lambda b: (0,) * len(shape))
    alpha_f, q, gff, gffs, valid = pl.pallas_call(
        _alpha_body,
        grid=(_B,),
        in_specs=[
            full((8, _D)), full((_D, _D)),
            full((1, _D)), full((1, _D)), full((1, 1)),
            pl.BlockSpec(memory_space=pltpu.SMEM),
            pl.BlockSpec(memory_space=pltpu.SMEM),
            pl.BlockSpec((1, _T, _D), lambda b: (b, 0, 0)),
        ],
        out_specs=[
            pl.BlockSpec((1, _FOLD, _D), lambda b: (b, 0, 0)),
            pl.BlockSpec((1, 1), lambda b: (0, 0)),
            pl.BlockSpec((1, 1, _NF), lambda b: (b, 0, 0)),
            pl.BlockSpec((1, 1, _NF), lambda b: (b, 0, 0)),
            pl.BlockSpec((1, _NF, 1), lambda b: (b, 0, 0)),
        ],
        out_shape=[
            jax.ShapeDtypeStruct((_B, _FOLD, _D), F32),
            jax.ShapeDtypeStruct((1, 1), F32),
            jax.ShapeDtypeStruct((_B, 1, _NF), I32),
            jax.ShapeDtypeStruct((_B, 1, _NF), I32),
            jax.ShapeDtypeStruct((_B, _NF, 1), F32),
        ],
    )(wk, dwT, db, pw, pb, input_lengths, target_lengths, fire_signal)

    sc_gather = pl.kernel(
        _sc_gather_body,
        mesh=plsc.VectorSubcoreMesh(core_axis_name="c", subcore_axis_name="s"),
        out_type=[
            jax.ShapeDtypeStruct((_NROWS, _D), F32),
            jax.ShapeDtypeStruct((_NROWS, _ACOP), F32),
        ],
        scratch_types=[
            pltpu.VMEM((_RPW,), I32),
            pltpu.VMEM((_RPW,), I32),
            pltpu.VMEM((_RPW, _D), F32),
            pltpu.VMEM((_RPW, _ACOP), F32),
            pltpu.SemaphoreType.DMA,
            pltpu.SemaphoreType.DMA,
        ],
    )
    temporal, pitch = sc_gather(
        gff.reshape(_NROWS), gffs.reshape(_NROWS),
        fire_signal.reshape(_B * _T, _D),
        jnp.pad(acoustic_src.reshape(_B * _TSW, _ACO),
                ((0, 0), (0, _ACOP - _ACO))))

    embs = pl.pallas_call(
        _comb_body,
        grid=(_B,),
        in_specs=[
            full((_D, _DM)), full((_ACOP, _DM)), full((1, _DM)),
            pl.BlockSpec((1, _NF, _D), lambda b: (b, 0, 0)),
            pl.BlockSpec((1, _NF, _ACOP), lambda b: (b, 0, 0)),
            pl.BlockSpec((1, _NF, 1), lambda b: (b, 0, 0)),
        ],
        out_specs=[pl.BlockSpec((1, _NF, _DM), lambda b: (b, 0, 0))],
        out_shape=[jax.ShapeDtypeStruct((_B, _NF, _DM), F32)],
    )(cw1T, cw2T, cb, temporal.reshape(_B, _NF, _D),
      pitch.reshape(_B, _NF, _ACOP), valid)[0]

    return embs, alpha_f.reshape(_B, _T), q[0, 0]


def kernel(fire_signal, acoustic_src, conv_w, ln_gamma, ln_beta, dense_w,
           dense_b, proj_w, proj_b, comb_w, comb_b, input_lengths,
           target_lengths):
    return _cif_run(fire_signal, acoustic_src, conv_w, ln_gamma, ln_beta,
                    dense_w, dense_b, proj_w, proj_b, comb_w, comb_b,
                    input_lengths, target_lengths)
